# baseline (speedup 1.0000x reference)
import functools

import numpy as np
import jax
import jax.numpy as jnp
from jax import lax
from jax.experimental import pallas as pl
from jax.experimental.pallas import tpu as pltpu
from jax.experimental.pallas import tpu_sc as plsc

B_, C_, H_, W_ = 4, 96, 224, 224
K_ = 32
G_ = 32
NW = 32
N_TOT = B_ * C_ * H_ * W_
PER_W = N_TOT // NW
ROWS_PER_W = (B_ * C_) // NW
ROW = H_ * W_
CHUNKS_PER_ROW = 4
CHUNK = ROW // CHUNKS_PER_ROW
N_CHUNKS = ROWS_PER_W * CHUNKS_PER_ROW
NJ = 48
TAB_N = C_ * 4 * NJ


def _build_tables(alpha, a, b, id_gain, bias):
    gidx = jnp.asarray((np.arange(C_) * G_) // C_, dtype=jnp.int32)
    apc = jnp.take(alpha, gidx, axis=0)
    iv = np.arange(NJ) - 8
    i0 = np.clip(iv - 1, 0, K_ - 1)
    i1 = np.clip(iv, 0, K_ - 1)
    i2 = np.clip(iv + 1, 0, K_ - 1)
    i3 = np.clip(iv + 2, 0, K_ - 1)
    a0, a1, a2, a3 = apc[:, i0], apc[:, i1], apc[:, i2], apc[:, i3]
    c0 = (a0 + 4.0 * a1 + a2) / 6.0 + bias[:, None]
    c1 = (a2 - a0) / 2.0
    c2 = (a0 - 2.0 * a1 + a2) / 2.0
    c3 = (3.0 * (a1 - a2) + (a3 - a0)) / 6.0
    tab = jnp.stack([c0, c1, c2, c3], axis=1).reshape(TAB_N).astype(jnp.float32)
    par = jnp.concatenate([15.5 * a, 15.5 * b + 23.5, id_gain]).astype(jnp.float32)
    return tab, par


_mesh = plsc.VectorSubcoreMesh(core_axis_name="c", subcore_axis_name="s")


@functools.partial(
    pl.kernel,
    mesh=_mesh,
    compiler_params=pltpu.CompilerParams(needs_layout_passes=False),
    out_type=jax.ShapeDtypeStruct((N_TOT,), jnp.float32),
    scratch_types=[
        pltpu.VMEM((TAB_N,), jnp.float32),
        pltpu.VMEM((3 * C_,), jnp.float32),
        pltpu.VMEM((CHUNK,), jnp.float32),
        pltpu.VMEM((CHUNK,), jnp.float32),
        pltpu.VMEM((CHUNK,), jnp.float32),
        pltpu.VMEM((CHUNK,), jnp.float32),
        pltpu.SemaphoreType.DMA,
        pltpu.SemaphoreType.DMA,
        pltpu.SemaphoreType.DMA,
        pltpu.SemaphoreType.DMA,
    ],
)
def _run(x_hbm, tab_hbm, par_hbm, out_hbm,
         tab_v, par_v, xin0, xin1, yout0, yout1,
         sin0, sin1, sout0, sout1):
    w = lax.axis_index("s") * 2 + lax.axis_index("c")
    pltpu.sync_copy(tab_hbm, tab_v)
    pltpu.sync_copy(par_hbm, par_v)
    base_w = w * PER_W
    xins = (xin0, xin1)
    youts = (yout0, yout1)
    sins = (sin0, sin1)
    souts = (sout0, sout1)

    def chunk_off(q):
        return base_w + q * CHUNK

    def start_in(q, slot):
        pltpu.async_copy(x_hbm.at[pl.ds(chunk_off(q), CHUNK)], xins[slot],
                         sins[slot])

    start_in(0, 0)

    def do_chunk(q, slot):
        row = w * ROWS_PER_W + q // CHUNKS_PER_ROW
        c = lax.rem(row, C_)
        cc = jnp.full((16,), c, dtype=jnp.int32)
        av = plsc.load_gather(par_v, [cc])
        bv = plsc.load_gather(par_v, [cc + C_])
        gv = plsc.load_gather(par_v, [cc + 2 * C_])
        tb = c * (4 * NJ)
        xin = xins[slot]
        yout = youts[slot]
        pltpu.make_async_copy(x_hbm.at[pl.ds(chunk_off(q), CHUNK)], xin,
                              sins[slot]).wait()
        @pl.when(q >= 2)
        def _():
            pltpu.make_async_copy(
                yout, out_hbm.at[pl.ds(chunk_off(q - 2), CHUNK)],
                souts[slot]).wait()
        @pl.when(q + 1 < N_CHUNKS)
        def _():
            pltpu.async_copy(x_hbm.at[pl.ds(chunk_off(q + 1), CHUNK)],
                             xins[1 - slot], sins[1 - slot])

        @plsc.parallel_loop(0, CHUNK // 16, unroll=8)
        def _body(i):
            s = i * 16
            xv = xin[pl.ds(s, 16)]
            u = jnp.minimum(jnp.maximum(xv * av + bv, 0.25), 46.75)
            ji = u.astype(jnp.int32)
            t = u - ji.astype(jnp.float32)
            idx = ji + tb
            c0v = plsc.load_gather(tab_v, [idx])
            c1v = plsc.load_gather(tab_v, [idx + NJ])
            c2v = plsc.load_gather(tab_v, [idx + 2 * NJ])
            c3v = plsc.load_gather(tab_v, [idx + 3 * NJ])
            sp = ((c3v * t + c2v) * t + c1v) * t + c0v
            yout[pl.ds(s, 16)] = gv * xv + sp
        pltpu.async_copy(yout, out_hbm.at[pl.ds(chunk_off(q), CHUNK)],
                         souts[slot])

    def loop_body(h, _):
        q0 = h * 2
        do_chunk(q0, 0)
        do_chunk(q0 + 1, 1)
        return 0

    lax.fori_loop(0, N_CHUNKS // 2, loop_body, 0, unroll=False)
    pltpu.make_async_copy(yout0, out_hbm.at[pl.ds(chunk_off(N_CHUNKS - 2),
                                                  CHUNK)], sout0).wait()
    pltpu.make_async_copy(yout1, out_hbm.at[pl.ds(chunk_off(N_CHUNKS - 1),
                                                  CHUNK)], sout1).wait()


def kernel(x, alpha, a, b, id_gain, bias):
    tab, par = _build_tables(alpha, a, b, id_gain, bias)
    y = _run(x.reshape(N_TOT), tab, par)
    return y.reshape(B_, C_, H_, W_)


# speedup vs baseline: 1.9010x; 1.9010x over previous
import functools

import numpy as np
import jax
import jax.numpy as jnp
from jax import lax
from jax.experimental import pallas as pl
from jax.experimental.pallas import tpu as pltpu
from jax.experimental.pallas import tpu_sc as plsc

B_, C_, H_, W_ = 4, 96, 224, 224
K_ = 32
G_ = 32
NW = 32
NROWS = B_ * C_ * H_
ROWS_PER_W = NROWS // NW
IMGS_PER_W = ROWS_PER_W // H_
RCH = 56
CHUNKS_PER_IMG = H_ // RCH
N_CHUNKS = IMGS_PER_W * CHUNKS_PER_IMG
NSEG = W_ // 16
NJ = 48
REP_N = 4 * NJ * 16


def _build_tables(alpha, a, b, id_gain, bias):
    gidx = jnp.asarray((np.arange(C_) * G_) // C_, dtype=jnp.int32)
    apc = jnp.take(alpha, gidx, axis=0)
    iv = np.arange(NJ) - 8
    i0 = np.clip(iv - 1, 0, K_ - 1)
    i1 = np.clip(iv, 0, K_ - 1)
    i2 = np.clip(iv + 1, 0, K_ - 1)
    i3 = np.clip(iv + 2, 0, K_ - 1)
    a0, a1, a2, a3 = apc[:, i0], apc[:, i1], apc[:, i2], apc[:, i3]
    c0 = (a0 + 4.0 * a1 + a2) / 6.0 + bias[:, None]
    c1 = (a2 - a0) / 2.0
    c2 = (a0 - 2.0 * a1 + a2) / 2.0
    c3 = (3.0 * (a1 - a2) + (a3 - a0)) / 6.0
    tab = jnp.stack([c0, c1, c2, c3], axis=1).astype(jnp.float32)
    rep = jnp.broadcast_to(tab[:, :, :, None], (C_, 4, NJ, 16)).reshape(C_ * REP_N)
    par = jnp.concatenate([15.5 * a, 15.5 * b + 23.5, id_gain]).astype(jnp.float32)
    return rep, par


_mesh = plsc.VectorSubcoreMesh(core_axis_name="c", subcore_axis_name="s")


@functools.partial(
    pl.kernel,
    mesh=_mesh,
    compiler_params=pltpu.CompilerParams(needs_layout_passes=False),
    out_type=jax.ShapeDtypeStruct((NROWS, W_), jnp.float32),
    scratch_types=[
        pltpu.VMEM((REP_N,), jnp.float32),
        pltpu.VMEM((3 * C_,), jnp.float32),
        pltpu.VMEM((RCH, W_), jnp.float32),
        pltpu.VMEM((RCH, W_), jnp.float32),
        pltpu.VMEM((RCH, W_), jnp.float32),
        pltpu.VMEM((RCH, W_), jnp.float32),
        pltpu.SemaphoreType.DMA,
        pltpu.SemaphoreType.DMA,
        pltpu.SemaphoreType.DMA,
        pltpu.SemaphoreType.DMA,
    ],
)
def _run(x_hbm, tab_hbm, par_hbm, out_hbm,
         tab_v, par_v, xin0, xin1, yout0, yout1,
         sin0, sin1, sout0, sout1):
    w = lax.axis_index("s") * 2 + lax.axis_index("c")
    pltpu.sync_copy(par_hbm, par_v)
    base_row = w * ROWS_PER_W
    lane = lax.iota(jnp.int32, 16)
    xins = (xin0, xin1)
    youts = (yout0, yout1)
    sins = (sin0, sin1)
    souts = (sout0, sout1)

    def chunk_row(q):
        return base_row + q * RCH

    pltpu.async_copy(x_hbm.at[pl.ds(chunk_row(0), RCH)], xins[0], sins[0])

    def do_chunk(q, slot):
        img = w * IMGS_PER_W + q // CHUNKS_PER_IMG
        c = lax.rem(img, C_)
        cc = jnp.full((16,), c, dtype=jnp.int32)
        av = plsc.load_gather(par_v, [cc])
        bv = plsc.load_gather(par_v, [cc + C_])
        gv = plsc.load_gather(par_v, [cc + 2 * C_])
        xin = xins[slot]
        yout = youts[slot]
        @pl.when(lax.rem(q, CHUNKS_PER_IMG) == 0)
        def _():
            pltpu.sync_copy(tab_hbm.at[pl.ds(c * REP_N, REP_N)], tab_v)
        pltpu.make_async_copy(x_hbm.at[pl.ds(chunk_row(q), RCH)], xin,
                              sins[slot]).wait()
        @pl.when(q >= 2)
        def _():
            pltpu.make_async_copy(
                yout, out_hbm.at[pl.ds(chunk_row(q - 2), RCH)],
                souts[slot]).wait()
        @pl.when(q + 1 < N_CHUNKS)
        def _():
            pltpu.async_copy(x_hbm.at[pl.ds(chunk_row(q + 1), RCH)],
                             xins[1 - slot], sins[1 - slot])

        @plsc.parallel_loop(0, RCH, unroll=1)
        def _body(r):
            for seg in range(NSEG):
                c0i = seg * 16
                xv = xin[r, pl.ds(c0i, 16)]
                u = jnp.minimum(jnp.maximum(xv * av + bv, 0.25), 46.75)
                ji = u.astype(jnp.int32)
                t = u - ji.astype(jnp.float32)
                idx = (ji << 4) + lane
                c0v = plsc.load_gather(tab_v, [idx])
                c1v = plsc.load_gather(tab_v, [idx + (NJ * 16)])
                c2v = plsc.load_gather(tab_v, [idx + (2 * NJ * 16)])
                c3v = plsc.load_gather(tab_v, [idx + (3 * NJ * 16)])
                sp = ((c3v * t + c2v) * t + c1v) * t + c0v
                yout[r, pl.ds(c0i, 16)] = gv * xv + sp

        pltpu.async_copy(yout, out_hbm.at[pl.ds(chunk_row(q), RCH)],
                         souts[slot])

    def loop_body(h, _):
        q0 = h * 2
        do_chunk(q0, 0)
        do_chunk(q0 + 1, 1)
        return 0

    lax.fori_loop(0, N_CHUNKS // 2, loop_body, 0, unroll=False)
    pltpu.make_async_copy(yout0, out_hbm.at[pl.ds(chunk_row(N_CHUNKS - 2),
                                                  RCH)], sout0).wait()
    pltpu.make_async_copy(yout1, out_hbm.at[pl.ds(chunk_row(N_CHUNKS - 1),
                                                  RCH)], sout1).wait()


def kernel(x, alpha, a, b, id_gain, bias):
    tab, par = _build_tables(alpha, a, b, id_gain, bias)
    y = _run(x.reshape(NROWS, W_), tab, par)
    return y.reshape(B_, C_, H_, W_)
